# baseline (speedup 1.0000x reference)
import functools

import jax
import jax.numpy as jnp
from jax import lax
from jax.experimental import pallas as pl
from jax.experimental.pallas import tpu as pltpu
from jax.experimental.pallas import tpu_sc as plsc

B, A, MAXNB, BONDS = 512, 120, 10, 512
AFD, BFD, H = 82, 6, 128
DEPTH = 3
AP = 128
SP = MAXNB * AP
NEG = -1e30

NW = 32
MPW = B // NW
LG = 16
NAC = AP // LG
H2 = H // 2
PR = 8
NPG = H2 // PR
M = 8
F32 = jnp.float32


def _mm(x, w):
    return jnp.dot(x, w, preferred_element_type=F32,
                   precision=lax.Precision.HIGHEST)


def _pack(x):
    a = x[0:H2].astype(jnp.bfloat16)
    b = x[H2:H].astype(jnp.bfloat16)
    au = lax.bitcast_convert_type(a, jnp.uint16).astype(jnp.uint32)
    bu = lax.bitcast_convert_type(b, jnp.uint16).astype(jnp.uint32)
    return lax.bitcast_convert_type(au | (bu << 16), jnp.int32)


def _tc_pre_body(iaT_ref, ibT_ref, WaT_ref, WU2aT_ref, WU2bT_ref, bU2c_ref,
                 WnbT_ref, bnbc_ref,
                 afT_ref, au2xT_ref, u2bT_ref, hbbT_ref):
    negpad = jnp.full((H, AP - A), NEG, F32)
    for m in range(M):
        afT = jnp.maximum(_mm(WaT_ref[...], iaT_ref[m]), 0.0)
        afT_ref[m, :, :] = afT
        au2T = _mm(WU2aT_ref[...], afT)
        au2xT_ref[m, :, :] = _pack(jnp.concatenate([au2T, negpad], axis=1))
        u2bT_ref[m, :, :] = _pack(_mm(WU2bT_ref[...], ibT_ref[m]) + bU2c_ref[...])
        hbbT_ref[m, :, :] = _pack(_mm(WnbT_ref[...], ibT_ref[m]) + bnbc_ref[...])


def _tc_mid_body(last, afT_ref, nlT_ref, nmT_ref,
                 WU1aT_ref, WU1bT_ref, bU1c_ref, WpT_ref, bpc_ref,
                 WsT_ref, bsc_ref, *out_refs):
    if last:
        pad = jnp.zeros((H, AP - A), F32)
    else:
        pad = jnp.full((H, AP - A), NEG, F32)
    for m in range(M):
        nlT = nlT_ref[m][:, 0:A]
        af2T = (_mm(WU1aT_ref[...], afT_ref[m]) + _mm(WU1bT_ref[...], nlT)
                + bU1c_ref[...])
        projT = _mm(WpT_ref[...], af2T) + bpc_ref[...]
        if not last:
            af_out_ref, projx_ref = out_refs
            af_out_ref[m, :, :] = af2T
        else:
            projx_ref, fsm_ref = out_refs
            fsm_ref[m, :, :] = (_mm(WsT_ref[...], af2T) + bsc_ref[...]) * nmT_ref[m]
        projx_ref[m, :, :] = _pack(jnp.concatenate([projT, pad], axis=1))


def _tc_post_body(fneiT_ref, fsmT_ref, WoT_ref, boc_ref, out_ref):
    outs = []
    for m in range(M):
        kernT = fneiT_ref[m][:, 0:A] * fsmT_ref[m]
        x = jnp.sum(kernT, axis=1, keepdims=True)
        outs.append(_mm(WoT_ref[...], x) + boc_ref[...])
    out_ref[...] = jnp.concatenate(outs, axis=0).reshape(M, 1, 1)


def _full(shape):
    return pl.BlockSpec(shape, lambda i: (0,) * len(shape))


def _perm(shape):
    return pl.BlockSpec((M,) + shape, lambda i: (i,) + (0,) * len(shape))


def _sc_layer(product, atabT, btabT, lan, lbn):
    mesh = plsc.VectorSubcoreMesh(core_axis_name="c", subcore_axis_name="s")

    @functools.partial(
        pl.kernel,
        out_type=jax.ShapeDtypeStruct((B * H, AP), F32),
        mesh=mesh,
        compiler_params=pltpu.CompilerParams(needs_layout_passes=False),
        scratch_types=[
            pltpu.VMEM((H2, AP), jnp.int32),
            pltpu.VMEM((H2, AP), jnp.int32),
            pltpu.VMEM((H2, BONDS), jnp.int32),
            pltpu.VMEM((H2, BONDS), jnp.int32),
            pltpu.VMEM((H, AP), F32),
            pltpu.VMEM((SP,), jnp.int32),
            pltpu.VMEM((SP,), jnp.int32),
            pltpu.VMEM((SP,), jnp.int32),
            pltpu.VMEM((SP,), jnp.int32),
            pltpu.SemaphoreType.DMA,
            pltpu.SemaphoreType.DMA,
            pltpu.SemaphoreType.DMA,
        ],
    )
    def body(atab_hbm, btab_hbm, lan_hbm, lbn_hbm, out_hbm,
             at0_v, at1_v, bt0_v, bt1_v, out_v, an0_v, bn0_v, an1_v, bn1_v,
             sem_at, sem_bt, sem_idx):
        wid = lax.axis_index("s") * 2 + lax.axis_index("c")
        mol_lo = wid * MPW
        mol_hi = mol_lo + MPW - 1

        def fetch_all(mol, at_buf, bt_buf, an_buf, bn_buf):
            a = pltpu.async_copy(atab_hbm.at[pl.ds(mol * H2, H2)], at_buf, sem_at)
            b = pltpu.async_copy(btab_hbm.at[pl.ds(mol * H2, H2)], bt_buf, sem_bt)
            i1 = pltpu.async_copy(lan_hbm.at[pl.ds(mol * SP, SP)], an_buf, sem_idx)
            i2 = pltpu.async_copy(lbn_hbm.at[pl.ds(mol * SP, SP)], bn_buf, sem_idx)
            return a, b, i1, i2

        def do_mol(mol, nxt, at_buf, bt_buf, an_buf, bn_buf,
                   atn_buf, btn_buf, ann_buf, bnn_buf):
            cps = fetch_all(nxt, atn_buf, btn_buf, ann_buf, bnn_buf)

            def per_achunk(ac, carry2):
                a0 = pl.multiple_of(ac * LG, LG)

                def per_pgroup(pg, carry3):
                    pvecs = [jnp.full((LG,), pg * PR + pi, jnp.int32)
                             for pi in range(PR)]
                    alo = [jnp.zeros((LG,), F32) for _ in range(PR)]
                    ahi = [jnp.zeros((LG,), F32) for _ in range(PR)]
                    for j in range(MAXNB):
                        aidx = an_buf[pl.ds(j * AP + a0, LG)]
                        bidx = bn_buf[pl.ds(j * AP + a0, LG)]
                        for pi in range(PR):
                            ga = plsc.load_gather(at_buf, [pvecs[pi], aidx])
                            gb = plsc.load_gather(bt_buf, [pvecs[pi], bidx])
                            al, ah = plsc.unpack(
                                plsc.bitcast(ga, jnp.bfloat16),
                                format=plsc.PackFormat.INTERLEAVED,
                                preferred_element_type=F32)
                            bl, bh = plsc.unpack(
                                plsc.bitcast(gb, jnp.bfloat16),
                                format=plsc.PackFormat.INTERLEAVED,
                                preferred_element_type=F32)
                            if product:
                                alo[pi] = alo[pi] + al * bl
                                ahi[pi] = ahi[pi] + ah * bh
                            else:
                                alo[pi] = alo[pi] + jnp.maximum(al + bl, 0.0)
                                ahi[pi] = ahi[pi] + jnp.maximum(ah + bh, 0.0)
                    for pi in range(PR):
                        out_v[pg * PR + pi, pl.ds(a0, LG)] = alo[pi]
                        out_v[pg * PR + pi + H2, pl.ds(a0, LG)] = ahi[pi]
                    return carry3

                return lax.fori_loop(0, NPG, per_pgroup, carry2)

            lax.fori_loop(0, NAC, per_achunk, 0)
            pltpu.sync_copy(out_v, out_hbm.at[pl.ds(mol * H, H)])
            for c in cps:
                c.wait()

        for c in fetch_all(mol_lo, at0_v, bt0_v, an0_v, bn0_v):
            c.wait()

        def per_pair(t, carry):
            molA = mol_lo + 2 * t
            molB = molA + 1
            nxtB = jnp.minimum(molB + 1, mol_hi)
            do_mol(molA, molB, at0_v, bt0_v, an0_v, bn0_v,
                   at1_v, bt1_v, an1_v, bn1_v)
            do_mol(molB, nxtB, at1_v, bt1_v, an1_v, bn1_v,
                   at0_v, bt0_v, an0_v, bn0_v)
            return carry

        lax.fori_loop(0, MPW // 2, per_pair, 0)

    return body(atabT, btabT, lan, lbn)


def kernel(input_atom, input_bond, atom_graph, bond_graph, num_nbs, node_mask,
           W_atom, W_nei_atom, b_nei_atom, W_nei_bond, b_nei_bond,
           W_self, b_self, W_U2, b_U2, W_U1, b_U1, W_out, b_out):
    i32 = jnp.int32
    a_nei = atom_graph[..., 1]
    b_nei = bond_graph[..., 1]
    mask = jnp.arange(MAXNB)[None, None, :] < num_nbs[:, :, None]
    an_safe = jnp.where(mask, a_nei, A)
    anT = jnp.transpose(an_safe, (0, 2, 1))
    bnT = jnp.transpose(b_nei, (0, 2, 1))
    apad = jnp.full((B, MAXNB, AP - A), A, anT.dtype)
    bpad = jnp.zeros((B, MAXNB, AP - A), bnT.dtype)
    lan = jnp.concatenate([anT, apad], axis=2).reshape(B * SP).astype(i32)
    lbn = jnp.concatenate([bnT, bpad], axis=2).reshape(B * SP).astype(i32)

    iaT = jnp.transpose(input_atom, (0, 2, 1))
    ibT = jnp.transpose(input_bond, (0, 2, 1))
    nmT = jnp.transpose(node_mask, (0, 2, 1))
    WaT = W_atom.T
    WU2aT, WU2bT = W_U2[:H].T, W_U2[H:].T
    WU1aT, WU1bT = W_U1[:H].T, W_U1[H:].T
    WnaT, WnbT, WsT = W_nei_atom.T, W_nei_bond.T, W_self.T
    col = lambda b: b.reshape(-1, 1).astype(F32)
    grid = (B // M,)

    afT, au2xT, u2bT, hbbT = pl.pallas_call(
        _tc_pre_body,
        grid=grid,
        in_specs=[_perm((AFD, A)), _perm((BFD, BONDS)),
                  _full((H, AFD)), _full((H, H)), _full((H, BFD)), _full((H, 1)),
                  _full((H, BFD)), _full((H, 1))],
        out_specs=[_perm((H, A)), _perm((H2, AP)), _perm((H2, BONDS)), _perm((H2, BONDS))],
        out_shape=[jax.ShapeDtypeStruct((B, H, A), F32),
                   jax.ShapeDtypeStruct((B, H2, AP), jnp.int32),
                   jax.ShapeDtypeStruct((B, H2, BONDS), jnp.int32),
                   jax.ShapeDtypeStruct((B, H2, BONDS), jnp.int32)],
    )(iaT, ibT, WaT, WU2aT, WU2bT, col(b_U2), WnbT, col(b_nei_bond))

    u2bT_f = u2bT.reshape(B * H2, BONDS)
    hbbT_f = hbbT.reshape(B * H2, BONDS)

    mid_specs = dict(
        grid=grid,
        in_specs=[_perm((H, A)), _perm((H, AP)), _perm((1, A)),
                  _full((H, H)), _full((H, H)), _full((H, 1)),
                  _full((H, H)), _full((H, 1)), _full((H, H)), _full((H, 1))],
    )

    af = afT
    aux = au2xT
    for depth in range(DEPTH - 1):
        last = depth == DEPTH - 2
        nl = _sc_layer(False, aux.reshape(B * H2, AP), u2bT_f, lan, lbn)
        nl = nl.reshape(B, H, AP)
        if not last:
            WpT, bp = WU2aT, jnp.zeros((H,), F32)
        else:
            WpT, bp = WnaT, b_nei_atom
        af, aux2 = pl.pallas_call(
            functools.partial(_tc_mid_body, last),
            out_specs=[_perm((H, A)) if not last else _perm((H2, AP)),
                       _perm((H2, AP)) if not last else _perm((H, A))],
            out_shape=[jax.ShapeDtypeStruct((B, H, A), F32) if not last
                       else jax.ShapeDtypeStruct((B, H2, AP), jnp.int32),
                       jax.ShapeDtypeStruct((B, H2, AP), jnp.int32) if not last
                       else jax.ShapeDtypeStruct((B, H, A), F32)],
            **mid_specs,
        )(af, nl, nmT, WU1aT, WU1bT, col(b_U1), WpT, col(bp), WsT, col(b_self))
        if not last:
            aux = aux2
        else:
            hax, fsm = af, aux2

    fnei = _sc_layer(True, hax.reshape(B * H2, AP), hbbT_f, lan, lbn)
    fnei = fnei.reshape(B, H, AP)

    out = pl.pallas_call(
        _tc_post_body,
        grid=grid,
        in_specs=[_perm((H, AP)), _perm((H, A)), _full((1, H)), _full((1, 1))],
        out_specs=pl.BlockSpec((M, 1, 1), lambda i: (i, 0, 0)),
        out_shape=jax.ShapeDtypeStruct((B, 1, 1), F32),
    )(fnei, fsm, W_out.T, col(b_out))
    return out.reshape(B, 1)


# speedup vs baseline: 1.2900x; 1.2900x over previous
import functools

import jax
import jax.numpy as jnp
from jax import lax
from jax.experimental import pallas as pl
from jax.experimental.pallas import tpu as pltpu
from jax.experimental.pallas import tpu_sc as plsc

B, A, MAXNB, BONDS = 512, 120, 10, 512
AFD, BFD, H = 82, 6, 128
DEPTH = 3
AP = 128
SP = MAXNB * AP
NEG = -1e30

NW = 32
MPW = B // NW
LG = 16
NAC = AP // LG
H2 = H // 2
PR = 8
NPG = H2 // PR
M = 8
F32 = jnp.float32


def _mm(x, w):
    return jnp.dot(x, w, preferred_element_type=F32)


def _bf16_rne(x):
    u = lax.bitcast_convert_type(x, jnp.uint32)
    return (u + 0x7FFF + ((u >> 16) & 1)) >> 16


def _pack(x):
    a = _bf16_rne(x[0:H2])
    b = _bf16_rne(x[H2:H])
    return lax.bitcast_convert_type(a | (b << 16), jnp.int32)


def _tc_pre_body(iaT_ref, ibT_ref, WaT_ref, WU2aT_ref, WU2bT_ref, bU2c_ref,
                 WnbT_ref, bnbc_ref,
                 afT_ref, au2xT_ref, u2bT_ref, hbbT_ref):
    negpad = jnp.full((H, AP - A), NEG, F32)
    for m in range(M):
        afT = jnp.maximum(_mm(WaT_ref[...], iaT_ref[m]), 0.0)
        afT_ref[m, :, :] = afT
        au2T = _mm(WU2aT_ref[...], afT)
        au2xT_ref[m, :, :] = _pack(jnp.concatenate([au2T, negpad], axis=1))
        u2bT_ref[m, :, :] = _pack(_mm(WU2bT_ref[...], ibT_ref[m]) + bU2c_ref[...])
        hbbT_ref[m, :, :] = _pack(_mm(WnbT_ref[...], ibT_ref[m]) + bnbc_ref[...])


def _tc_mid_body(last, afT_ref, nlT_ref, nmT_ref,
                 WU1aT_ref, WU1bT_ref, bU1c_ref, WpT_ref, bpc_ref,
                 WsT_ref, bsc_ref, *out_refs):
    if last:
        pad = jnp.zeros((H, AP - A), F32)
    else:
        pad = jnp.full((H, AP - A), NEG, F32)
    for m in range(M):
        nlT = nlT_ref[m][:, 0:A]
        af2T = (_mm(WU1aT_ref[...], afT_ref[m]) + _mm(WU1bT_ref[...], nlT)
                + bU1c_ref[...])
        projT = _mm(WpT_ref[...], af2T) + bpc_ref[...]
        if not last:
            af_out_ref, projx_ref = out_refs
            af_out_ref[m, :, :] = af2T
        else:
            projx_ref, fsm_ref = out_refs
            fsm_ref[m, :, :] = (_mm(WsT_ref[...], af2T) + bsc_ref[...]) * nmT_ref[m]
        projx_ref[m, :, :] = _pack(jnp.concatenate([projT, pad], axis=1))


def _tc_post_body(fneiT_ref, fsmT_ref, WoT_ref, boc_ref, out_ref):
    outs = []
    for m in range(M):
        kernT = fneiT_ref[m][:, 0:A] * fsmT_ref[m]
        x = jnp.sum(kernT, axis=1, keepdims=True)
        outs.append(_mm(WoT_ref[...], x) + boc_ref[...])
    out_ref[...] = jnp.concatenate(outs, axis=0).reshape(M, 1, 1)


def _full(shape):
    return pl.BlockSpec(shape, lambda i: (0,) * len(shape))


def _perm(shape):
    return pl.BlockSpec((M,) + shape, lambda i: (i,) + (0,) * len(shape))


def _sc_layer(product, atabT, btabT, lan, lbn):
    mesh = plsc.VectorSubcoreMesh(core_axis_name="c", subcore_axis_name="s")

    @functools.partial(
        pl.kernel,
        out_type=jax.ShapeDtypeStruct((B * H, AP), F32),
        mesh=mesh,
        compiler_params=pltpu.CompilerParams(needs_layout_passes=False),
        scratch_types=[
            pltpu.VMEM((H2, AP), jnp.int32),
            pltpu.VMEM((H2, AP), jnp.int32),
            pltpu.VMEM((H2, BONDS), jnp.int32),
            pltpu.VMEM((H2, BONDS), jnp.int32),
            pltpu.VMEM((H, AP), F32),
            pltpu.VMEM((SP,), jnp.int32),
            pltpu.VMEM((SP,), jnp.int32),
            pltpu.VMEM((SP,), jnp.int32),
            pltpu.VMEM((SP,), jnp.int32),
            pltpu.SemaphoreType.DMA,
            pltpu.SemaphoreType.DMA,
            pltpu.SemaphoreType.DMA,
        ],
    )
    def body(atab_hbm, btab_hbm, lan_hbm, lbn_hbm, out_hbm,
             at0_v, at1_v, bt0_v, bt1_v, out_v, an0_v, bn0_v, an1_v, bn1_v,
             sem_at, sem_bt, sem_idx):
        wid = lax.axis_index("s") * 2 + lax.axis_index("c")
        mol_lo = wid * MPW
        mol_hi = mol_lo + MPW - 1

        def fetch_all(mol, at_buf, bt_buf, an_buf, bn_buf):
            a = pltpu.async_copy(atab_hbm.at[pl.ds(mol * H2, H2)], at_buf, sem_at)
            b = pltpu.async_copy(btab_hbm.at[pl.ds(mol * H2, H2)], bt_buf, sem_bt)
            i1 = pltpu.async_copy(lan_hbm.at[pl.ds(mol * SP, SP)], an_buf, sem_idx)
            i2 = pltpu.async_copy(lbn_hbm.at[pl.ds(mol * SP, SP)], bn_buf, sem_idx)
            return a, b, i1, i2

        def do_mol(mol, nxt, at_buf, bt_buf, an_buf, bn_buf,
                   atn_buf, btn_buf, ann_buf, bnn_buf):
            cps = fetch_all(nxt, atn_buf, btn_buf, ann_buf, bnn_buf)

            def per_achunk(ac, carry2):
                a0 = pl.multiple_of(ac * LG, LG)

                def per_pgroup(pg, carry3):
                    pvecs = [jnp.full((LG,), pg * PR + pi, jnp.int32)
                             for pi in range(PR)]
                    alo = [jnp.zeros((LG,), F32) for _ in range(PR)]
                    ahi = [jnp.zeros((LG,), F32) for _ in range(PR)]
                    for j in range(MAXNB):
                        aidx = an_buf[pl.ds(j * AP + a0, LG)]
                        bidx = bn_buf[pl.ds(j * AP + a0, LG)]
                        for pi in range(PR):
                            ga = plsc.load_gather(at_buf, [pvecs[pi], aidx])
                            gb = plsc.load_gather(bt_buf, [pvecs[pi], bidx])
                            al, ah = plsc.unpack(
                                plsc.bitcast(ga, jnp.bfloat16),
                                format=plsc.PackFormat.INTERLEAVED,
                                preferred_element_type=F32)
                            bl, bh = plsc.unpack(
                                plsc.bitcast(gb, jnp.bfloat16),
                                format=plsc.PackFormat.INTERLEAVED,
                                preferred_element_type=F32)
                            if product:
                                alo[pi] = alo[pi] + al * bl
                                ahi[pi] = ahi[pi] + ah * bh
                            else:
                                alo[pi] = alo[pi] + jnp.maximum(al + bl, 0.0)
                                ahi[pi] = ahi[pi] + jnp.maximum(ah + bh, 0.0)
                    for pi in range(PR):
                        out_v[pg * PR + pi, pl.ds(a0, LG)] = alo[pi]
                        out_v[pg * PR + pi + H2, pl.ds(a0, LG)] = ahi[pi]
                    return carry3

                return lax.fori_loop(0, NPG, per_pgroup, carry2)

            lax.fori_loop(0, NAC, per_achunk, 0)
            pltpu.sync_copy(out_v, out_hbm.at[pl.ds(mol * H, H)])
            for c in cps:
                c.wait()

        for c in fetch_all(mol_lo, at0_v, bt0_v, an0_v, bn0_v):
            c.wait()

        def per_pair(t, carry):
            molA = mol_lo + 2 * t
            molB = molA + 1
            nxtB = jnp.minimum(molB + 1, mol_hi)
            do_mol(molA, molB, at0_v, bt0_v, an0_v, bn0_v,
                   at1_v, bt1_v, an1_v, bn1_v)
            do_mol(molB, nxtB, at1_v, bt1_v, an1_v, bn1_v,
                   at0_v, bt0_v, an0_v, bn0_v)
            return carry

        lax.fori_loop(0, MPW // 2, per_pair, 0)

    return body(atabT, btabT, lan, lbn)


def kernel(input_atom, input_bond, atom_graph, bond_graph, num_nbs, node_mask,
           W_atom, W_nei_atom, b_nei_atom, W_nei_bond, b_nei_bond,
           W_self, b_self, W_U2, b_U2, W_U1, b_U1, W_out, b_out):
    i32 = jnp.int32
    a_nei = atom_graph[..., 1]
    b_nei = bond_graph[..., 1]
    mask = jnp.arange(MAXNB)[None, None, :] < num_nbs[:, :, None]
    an_safe = jnp.where(mask, a_nei, A)
    anT = jnp.transpose(an_safe, (0, 2, 1))
    bnT = jnp.transpose(b_nei, (0, 2, 1))
    apad = jnp.full((B, MAXNB, AP - A), A, anT.dtype)
    bpad = jnp.zeros((B, MAXNB, AP - A), bnT.dtype)
    lan = jnp.concatenate([anT, apad], axis=2).reshape(B * SP).astype(i32)
    lbn = jnp.concatenate([bnT, bpad], axis=2).reshape(B * SP).astype(i32)

    iaT = jnp.transpose(input_atom, (0, 2, 1))
    ibT = jnp.transpose(input_bond, (0, 2, 1))
    nmT = jnp.transpose(node_mask, (0, 2, 1))
    WaT = W_atom.T
    WU2aT, WU2bT = W_U2[:H].T, W_U2[H:].T
    WU1aT, WU1bT = W_U1[:H].T, W_U1[H:].T
    WnaT, WnbT, WsT = W_nei_atom.T, W_nei_bond.T, W_self.T
    col = lambda b: b.reshape(-1, 1).astype(F32)
    grid = (B // M,)

    afT, au2xT, u2bT, hbbT = pl.pallas_call(
        _tc_pre_body,
        grid=grid,
        in_specs=[_perm((AFD, A)), _perm((BFD, BONDS)),
                  _full((H, AFD)), _full((H, H)), _full((H, BFD)), _full((H, 1)),
                  _full((H, BFD)), _full((H, 1))],
        out_specs=[_perm((H, A)), _perm((H2, AP)), _perm((H2, BONDS)), _perm((H2, BONDS))],
        out_shape=[jax.ShapeDtypeStruct((B, H, A), F32),
                   jax.ShapeDtypeStruct((B, H2, AP), jnp.int32),
                   jax.ShapeDtypeStruct((B, H2, BONDS), jnp.int32),
                   jax.ShapeDtypeStruct((B, H2, BONDS), jnp.int32)],
    )(iaT, ibT, WaT, WU2aT, WU2bT, col(b_U2), WnbT, col(b_nei_bond))

    u2bT_f = u2bT.reshape(B * H2, BONDS)
    hbbT_f = hbbT.reshape(B * H2, BONDS)

    mid_specs = dict(
        grid=grid,
        in_specs=[_perm((H, A)), _perm((H, AP)), _perm((1, A)),
                  _full((H, H)), _full((H, H)), _full((H, 1)),
                  _full((H, H)), _full((H, 1)), _full((H, H)), _full((H, 1))],
    )

    af = afT
    aux = au2xT
    for depth in range(DEPTH - 1):
        last = depth == DEPTH - 2
        nl = _sc_layer(False, aux.reshape(B * H2, AP), u2bT_f, lan, lbn)
        nl = nl.reshape(B, H, AP)
        if not last:
            WpT, bp = WU2aT, jnp.zeros((H,), F32)
        else:
            WpT, bp = WnaT, b_nei_atom
        af, aux2 = pl.pallas_call(
            functools.partial(_tc_mid_body, last),
            out_specs=[_perm((H, A)) if not last else _perm((H2, AP)),
                       _perm((H2, AP)) if not last else _perm((H, A))],
            out_shape=[jax.ShapeDtypeStruct((B, H, A), F32) if not last
                       else jax.ShapeDtypeStruct((B, H2, AP), jnp.int32),
                       jax.ShapeDtypeStruct((B, H2, AP), jnp.int32) if not last
                       else jax.ShapeDtypeStruct((B, H, A), F32)],
            **mid_specs,
        )(af, nl, nmT, WU1aT, WU1bT, col(b_U1), WpT, col(bp), WsT, col(b_self))
        if not last:
            aux = aux2
        else:
            hax, fsm = af, aux2

    fnei = _sc_layer(True, hax.reshape(B * H2, AP), hbbT_f, lan, lbn)
    fnei = fnei.reshape(B, H, AP)

    out = pl.pallas_call(
        _tc_post_body,
        grid=grid,
        in_specs=[_perm((H, AP)), _perm((H, A)), _full((1, H)), _full((1, 1))],
        out_specs=pl.BlockSpec((M, 1, 1), lambda i: (i, 0, 0)),
        out_shape=jax.ShapeDtypeStruct((B, 1, 1), F32),
    )(fnei, fsm, W_out.T, col(b_out))
    return out.reshape(B, 1)


# speedup vs baseline: 1.3473x; 1.0444x over previous
import functools

import jax
import jax.numpy as jnp
from jax import lax
from jax.experimental import pallas as pl
from jax.experimental.pallas import tpu as pltpu
from jax.experimental.pallas import tpu_sc as plsc

B, A, MAXNB, BONDS = 512, 120, 10, 512
AFD, BFD, H = 82, 6, 128
DEPTH = 3
AP = 128
SP = MAXNB * AP
NEG = -1e30

NW = 32
MPW = B // NW
LG = 16
NAC = AP // LG
H2 = H // 2
PR = 8
NPG = H2 // PR
M = 16
F32 = jnp.float32


def _mm(x, w):
    return jnp.dot(x, w, preferred_element_type=F32)


def _bf16_rne(x):
    u = lax.bitcast_convert_type(x, jnp.uint32)
    return (u + 0x7FFF + ((u >> 16) & 1)) >> 16


def _pack(x):
    a = _bf16_rne(x[0:H2])
    b = _bf16_rne(x[H2:H])
    return lax.bitcast_convert_type(a | (b << 16), jnp.int32)


def _tc_pre_body(iaT_ref, ibT_ref, WaT_ref, WU2aT_ref, WU2bT_ref, bU2c_ref,
                 WnbT_ref, bnbc_ref,
                 afT_ref, au2xT_ref, u2bT_ref, hbbT_ref):
    negpad = jnp.full((H, AP - A), NEG, F32)
    for m in range(M):
        afT = jnp.maximum(_mm(WaT_ref[...], iaT_ref[m]), 0.0)
        afT_ref[m, :, :] = afT
        au2T = _mm(WU2aT_ref[...], afT)
        au2xT_ref[m, :, :] = _pack(jnp.concatenate([au2T, negpad], axis=1))
        u2bT_ref[m, :, :] = _pack(_mm(WU2bT_ref[...], ibT_ref[m]) + bU2c_ref[...])
        hbbT_ref[m, :, :] = _pack(_mm(WnbT_ref[...], ibT_ref[m]) + bnbc_ref[...])


def _tc_mid_body(last, afT_ref, nlT_ref, nmT_ref,
                 WU1aT_ref, WU1bT_ref, bU1c_ref, WpT_ref, bpc_ref,
                 WsT_ref, bsc_ref, *out_refs):
    if last:
        pad = jnp.zeros((H, AP - A), F32)
    else:
        pad = jnp.full((H, AP - A), NEG, F32)
    for m in range(M):
        nlT = nlT_ref[m][:, 0:A]
        af2T = (_mm(WU1aT_ref[...], afT_ref[m]) + _mm(WU1bT_ref[...], nlT)
                + bU1c_ref[...])
        projT = _mm(WpT_ref[...], af2T) + bpc_ref[...]
        if not last:
            af_out_ref, projx_ref = out_refs
            af_out_ref[m, :, :] = af2T
        else:
            projx_ref, fsm_ref = out_refs
            fsm_ref[m, :, :] = (_mm(WsT_ref[...], af2T) + bsc_ref[...]) * nmT_ref[m]
        projx_ref[m, :, :] = _pack(jnp.concatenate([projT, pad], axis=1))


def _tc_post_body(fneiT_ref, fsmT_ref, WoT_ref, boc_ref, out_ref):
    outs = []
    for m in range(M):
        kernT = fneiT_ref[m][:, 0:A] * fsmT_ref[m]
        x = jnp.sum(kernT, axis=1, keepdims=True)
        outs.append(_mm(WoT_ref[...], x) + boc_ref[...])
    out_ref[...] = jnp.concatenate(outs, axis=0).reshape(M, 1, 1)


def _full(shape):
    return pl.BlockSpec(shape, lambda i: (0,) * len(shape))


def _perm(shape):
    return pl.BlockSpec((M,) + shape, lambda i: (i,) + (0,) * len(shape))


def _sc_layer(product, atabT, btabT, lan, lbn):
    mesh = plsc.VectorSubcoreMesh(core_axis_name="c", subcore_axis_name="s")

    @functools.partial(
        pl.kernel,
        out_type=jax.ShapeDtypeStruct((B * H, AP), F32),
        mesh=mesh,
        compiler_params=pltpu.CompilerParams(needs_layout_passes=False),
        scratch_types=[
            pltpu.VMEM((H2, AP), jnp.int32),
            pltpu.VMEM((H2, AP), jnp.int32),
            pltpu.VMEM((H2, BONDS), jnp.int32),
            pltpu.VMEM((H2, BONDS), jnp.int32),
            pltpu.VMEM((H, AP), F32),
            pltpu.VMEM((SP,), jnp.int32),
            pltpu.VMEM((SP,), jnp.int32),
            pltpu.VMEM((SP,), jnp.int32),
            pltpu.VMEM((SP,), jnp.int32),
            pltpu.SemaphoreType.DMA,
            pltpu.SemaphoreType.DMA,
            pltpu.SemaphoreType.DMA,
        ],
    )
    def body(atab_hbm, btab_hbm, lan_hbm, lbn_hbm, out_hbm,
             at0_v, at1_v, bt0_v, bt1_v, out_v, an0_v, bn0_v, an1_v, bn1_v,
             sem_at, sem_bt, sem_idx):
        wid = lax.axis_index("s") * 2 + lax.axis_index("c")
        mol_lo = wid * MPW
        mol_hi = mol_lo + MPW - 1

        def fetch_all(mol, at_buf, bt_buf, an_buf, bn_buf):
            a = pltpu.async_copy(atab_hbm.at[pl.ds(mol * H2, H2)], at_buf, sem_at)
            b = pltpu.async_copy(btab_hbm.at[pl.ds(mol * H2, H2)], bt_buf, sem_bt)
            i1 = pltpu.async_copy(lan_hbm.at[pl.ds(mol * SP, SP)], an_buf, sem_idx)
            i2 = pltpu.async_copy(lbn_hbm.at[pl.ds(mol * SP, SP)], bn_buf, sem_idx)
            return a, b, i1, i2

        def do_mol(mol, nxt, at_buf, bt_buf, an_buf, bn_buf,
                   atn_buf, btn_buf, ann_buf, bnn_buf):
            cps = fetch_all(nxt, atn_buf, btn_buf, ann_buf, bnn_buf)

            def per_achunk(ac, carry2):
                a0 = pl.multiple_of(ac * LG, LG)

                def per_pgroup(pg, carry3):
                    pvecs = [jnp.full((LG,), pg * PR + pi, jnp.int32)
                             for pi in range(PR)]
                    alo = [jnp.zeros((LG,), F32) for _ in range(PR)]
                    ahi = [jnp.zeros((LG,), F32) for _ in range(PR)]
                    for j in range(MAXNB):
                        aidx = an_buf[pl.ds(j * AP + a0, LG)]
                        bidx = bn_buf[pl.ds(j * AP + a0, LG)]
                        for pi in range(PR):
                            ga = plsc.load_gather(at_buf, [pvecs[pi], aidx])
                            gb = plsc.load_gather(bt_buf, [pvecs[pi], bidx])
                            al, ah = plsc.unpack(
                                plsc.bitcast(ga, jnp.bfloat16),
                                format=plsc.PackFormat.INTERLEAVED,
                                preferred_element_type=F32)
                            bl, bh = plsc.unpack(
                                plsc.bitcast(gb, jnp.bfloat16),
                                format=plsc.PackFormat.INTERLEAVED,
                                preferred_element_type=F32)
                            if product:
                                alo[pi] = alo[pi] + al * bl
                                ahi[pi] = ahi[pi] + ah * bh
                            else:
                                alo[pi] = alo[pi] + jnp.maximum(al + bl, 0.0)
                                ahi[pi] = ahi[pi] + jnp.maximum(ah + bh, 0.0)
                    for pi in range(PR):
                        out_v[pg * PR + pi, pl.ds(a0, LG)] = alo[pi]
                        out_v[pg * PR + pi + H2, pl.ds(a0, LG)] = ahi[pi]
                    return carry3

                return lax.fori_loop(0, NPG, per_pgroup, carry2)

            lax.fori_loop(0, NAC, per_achunk, 0)
            pltpu.sync_copy(out_v, out_hbm.at[pl.ds(mol * H, H)])
            for c in cps:
                c.wait()

        for c in fetch_all(mol_lo, at0_v, bt0_v, an0_v, bn0_v):
            c.wait()

        def per_pair(t, carry):
            molA = mol_lo + 2 * t
            molB = molA + 1
            nxtB = jnp.minimum(molB + 1, mol_hi)
            do_mol(molA, molB, at0_v, bt0_v, an0_v, bn0_v,
                   at1_v, bt1_v, an1_v, bn1_v)
            do_mol(molB, nxtB, at1_v, bt1_v, an1_v, bn1_v,
                   at0_v, bt0_v, an0_v, bn0_v)
            return carry

        lax.fori_loop(0, MPW // 2, per_pair, 0)

    return body(atabT, btabT, lan, lbn)


def kernel(input_atom, input_bond, atom_graph, bond_graph, num_nbs, node_mask,
           W_atom, W_nei_atom, b_nei_atom, W_nei_bond, b_nei_bond,
           W_self, b_self, W_U2, b_U2, W_U1, b_U1, W_out, b_out):
    i32 = jnp.int32
    a_nei = atom_graph[..., 1]
    b_nei = bond_graph[..., 1]
    mask = jnp.arange(MAXNB)[None, None, :] < num_nbs[:, :, None]
    an_safe = jnp.where(mask, a_nei, A)
    anT = jnp.transpose(an_safe, (0, 2, 1))
    bnT = jnp.transpose(b_nei, (0, 2, 1))
    apad = jnp.full((B, MAXNB, AP - A), A, anT.dtype)
    bpad = jnp.zeros((B, MAXNB, AP - A), bnT.dtype)
    lan = jnp.concatenate([anT, apad], axis=2).reshape(B * SP).astype(i32)
    lbn = jnp.concatenate([bnT, bpad], axis=2).reshape(B * SP).astype(i32)

    iaT = jnp.transpose(input_atom, (0, 2, 1))
    ibT = jnp.transpose(input_bond, (0, 2, 1))
    nmT = jnp.transpose(node_mask, (0, 2, 1))
    WaT = W_atom.T
    WU2aT, WU2bT = W_U2[:H].T, W_U2[H:].T
    WU1aT, WU1bT = W_U1[:H].T, W_U1[H:].T
    WnaT, WnbT, WsT = W_nei_atom.T, W_nei_bond.T, W_self.T
    col = lambda b: b.reshape(-1, 1).astype(F32)
    grid = (B // M,)

    afT, au2xT, u2bT, hbbT = pl.pallas_call(
        _tc_pre_body,
        grid=grid,
        in_specs=[_perm((AFD, A)), _perm((BFD, BONDS)),
                  _full((H, AFD)), _full((H, H)), _full((H, BFD)), _full((H, 1)),
                  _full((H, BFD)), _full((H, 1))],
        out_specs=[_perm((H, A)), _perm((H2, AP)), _perm((H2, BONDS)), _perm((H2, BONDS))],
        out_shape=[jax.ShapeDtypeStruct((B, H, A), F32),
                   jax.ShapeDtypeStruct((B, H2, AP), jnp.int32),
                   jax.ShapeDtypeStruct((B, H2, BONDS), jnp.int32),
                   jax.ShapeDtypeStruct((B, H2, BONDS), jnp.int32)],
    )(iaT, ibT, WaT, WU2aT, WU2bT, col(b_U2), WnbT, col(b_nei_bond))

    u2bT_f = u2bT.reshape(B * H2, BONDS)
    hbbT_f = hbbT.reshape(B * H2, BONDS)

    mid_specs = dict(
        grid=grid,
        in_specs=[_perm((H, A)), _perm((H, AP)), _perm((1, A)),
                  _full((H, H)), _full((H, H)), _full((H, 1)),
                  _full((H, H)), _full((H, 1)), _full((H, H)), _full((H, 1))],
    )

    af = afT
    aux = au2xT
    for depth in range(DEPTH - 1):
        last = depth == DEPTH - 2
        nl = _sc_layer(False, aux.reshape(B * H2, AP), u2bT_f, lan, lbn)
        nl = nl.reshape(B, H, AP)
        if not last:
            WpT, bp = WU2aT, jnp.zeros((H,), F32)
        else:
            WpT, bp = WnaT, b_nei_atom
        af, aux2 = pl.pallas_call(
            functools.partial(_tc_mid_body, last),
            out_specs=[_perm((H, A)) if not last else _perm((H2, AP)),
                       _perm((H2, AP)) if not last else _perm((H, A))],
            out_shape=[jax.ShapeDtypeStruct((B, H, A), F32) if not last
                       else jax.ShapeDtypeStruct((B, H2, AP), jnp.int32),
                       jax.ShapeDtypeStruct((B, H2, AP), jnp.int32) if not last
                       else jax.ShapeDtypeStruct((B, H, A), F32)],
            **mid_specs,
        )(af, nl, nmT, WU1aT, WU1bT, col(b_U1), WpT, col(bp), WsT, col(b_self))
        if not last:
            aux = aux2
        else:
            hax, fsm = af, aux2

    fnei = _sc_layer(True, hax.reshape(B * H2, AP), hbbT_f, lan, lbn)
    fnei = fnei.reshape(B, H, AP)

    out = pl.pallas_call(
        _tc_post_body,
        grid=grid,
        in_specs=[_perm((H, AP)), _perm((H, A)), _full((1, H)), _full((1, 1))],
        out_specs=pl.BlockSpec((M, 1, 1), lambda i: (i, 0, 0)),
        out_shape=jax.ShapeDtypeStruct((B, 1, 1), F32),
    )(fnei, fsm, W_out.T, col(b_out))
    return out.reshape(B, 1)
